# Initial kernel scaffold; baseline (speedup 1.0000x reference)
import jax
import jax.numpy as jnp
from jax.experimental import pallas as pl


def kernel(pos, rgb, batch, W1a, b1a, W2a, b2a, W1b, b1b, W2b, b2b, W1c, b1c, W2c, b2c):
    raise NotImplementedError("write your pallas kernel here")


# speedup vs baseline: 8.5691x; 8.5691x over previous
import functools

import jax
import jax.numpy as jnp
from jax.experimental import pallas as pl
from jax.experimental.pallas import tpu as pltpu

RADIUS2 = (1.0 / 16.0) ** 2
ROWS = 32

CB = 3
CS = 4


def _layer_body(pos5nd_ref, pos5T_ref, xT_ref, W1_ref, b1_ref, W2_ref, b2_ref,
                out_ref, A_ref, B_ref, agg_ref,
                *, c_in, c_mid, nb, last):
    pid = pl.program_id(0)

    @pl.when(pid == 0)
    def _prep():
        W1x16 = W1_ref[0:c_in, :].astype(jnp.bfloat16)
        W1r32 = W1_ref[c_in:c_in + 3, :].astype(jnp.bfloat16).astype(jnp.float32)
        a = jax.lax.dot_general(W1x16, xT_ref[...].astype(jnp.bfloat16),
                                (((0,), (0,)), ((), ())),
                                preferred_element_type=jnp.float32)
        a = a + jax.lax.dot_general(W1r32, pos5T_ref[0:3, :],
                                    (((0,), (0,)), ((), ())),
                                    precision=jax.lax.Precision.HIGHEST,
                                    preferred_element_type=jnp.float32)
        A_ref[0:c_mid, :] = a + b1_ref[...]
        B_ref[:, 0:c_mid] = jnp.dot(pos5nd_ref[...][:, 0:3], W1r32,
                                    precision=jax.lax.Precision.HIGHEST,
                                    preferred_element_type=jnp.float32)

    @pl.when((pid >= 1) & (pid <= nb))
    def _main():
        i0 = (pid - 1) * ROWS
        P = pos5nd_ref[pl.ds(i0, ROWS), :]
        dot = jax.lax.dot_general(P[:, 0:3], pos5T_ref[0:3, :],
                                  (((1,), (0,)), ((), ())),
                                  preferred_element_type=jnp.float32)
        d2 = pos5T_ref[CS:CS + 1, :] + P[:, CS:CS + 1] - 2.0 * dot
        valid = jnp.logical_and(d2 <= RADIUS2,
                                P[:, CB:CB + 1] == pos5T_ref[CB:CB + 1, :])
        Brows = B_ref[pl.ds(i0, ROWS), 0:c_mid]
        cols = []
        for c in range(c_mid):
            h = jnp.maximum(A_ref[c:c + 1, :] - Brows[:, c:c + 1], 0.0)
            hm = jnp.where(valid, h, 0.0)
            cols.append(jnp.max(hm, axis=1, keepdims=True))
        agg_ref[pl.ds(i0, ROWS), 0:c_mid] = jnp.concatenate(cols, axis=1)

    @pl.when(pid == nb + 1)
    def _epilogue():
        agg16 = agg_ref[...][:, 0:c_mid].astype(jnp.bfloat16)
        W2_16 = W2_ref[...].astype(jnp.bfloat16)
        if last:
            y = jnp.dot(agg16, W2_16,
                        preferred_element_type=jnp.float32) + b2_ref[...]
        else:
            y = jax.lax.dot_general(W2_16, agg16, (((0,), (1,)), ((), ())),
                                    preferred_element_type=jnp.float32)
            y = y + b2_ref[...]
        out_ref[...] = jnp.where(y > 0, y, jnp.exp(jnp.minimum(y, 0.0)) - 1.0)


def _layer(pos5nd, pos5T, xT, W1, b1, W2, b2, *, c_in, c_mid, c_out, last):
    np_ = pos5nd.shape[0]
    nb = np_ // ROWS
    if last:
        out_shape = jax.ShapeDtypeStruct((np_, c_out), jnp.float32)
        b2s = b2.reshape(1, c_out)
    else:
        out_shape = jax.ShapeDtypeStruct((c_out, np_), jnp.float32)
        b2s = b2.reshape(c_out, 1)
    body = functools.partial(_layer_body, c_in=c_in, c_mid=c_mid, nb=nb,
                             last=last)
    full = lambda s: pl.BlockSpec(s, lambda i: (0,) * len(s))
    return pl.pallas_call(
        body,
        grid=(nb + 2,),
        in_specs=[full(pos5nd.shape), full(pos5T.shape), full(xT.shape),
                  full(W1.shape), full((c_mid, 1)), full(W2.shape),
                  full(b2s.shape)],
        out_specs=full(out_shape.shape),
        out_shape=out_shape,
        scratch_shapes=[
            pltpu.VMEM((32, np_), jnp.float32),
            pltpu.VMEM((np_, 32), jnp.float32),
            pltpu.VMEM((np_, 32), jnp.float32),
        ],
    )(pos5nd, pos5T, xT, W1, b1.reshape(c_mid, 1), W2, b2s)


def kernel(pos, rgb, batch, W1a, b1a, W2a, b2a, W1b, b1b, W2b, b2b,
           W1c, b1c, W2c, b2c):
    n = pos.shape[0]
    np_ = ((n + 255) // 256) * 256
    pad = np_ - n
    pos_p = jnp.pad(pos, ((0, pad), (0, 0)), constant_values=2.0)
    b_p = jnp.pad(batch.astype(jnp.float32), (0, pad), constant_values=-1.0)
    sq = jnp.sum(pos_p * pos_p, axis=-1)
    pos5 = jnp.concatenate([pos_p, b_p[:, None], sq[:, None]], axis=1)
    pos5T = pos5.T

    xT = _layer(pos5, pos5T, pos5T[0:3, :], W1a, b1a, W2a, b2a,
                c_in=3, c_mid=8, c_out=8, last=False)
    xT = _layer(pos5, pos5T, xT, W1b, b1b, W2b, b2b,
                c_in=8, c_mid=16, c_out=16, last=False)
    out = _layer(pos5, pos5T, xT, W1c, b1c, W2c, b2c,
                 c_in=16, c_mid=32, c_out=32, last=True)
    return (pos, out[:n], batch)
